# Initial kernel scaffold; baseline (speedup 1.0000x reference)
import jax
import jax.numpy as jnp
from jax.experimental import pallas as pl


def kernel(x, edge_index, batch, v, lin):
    raise NotImplementedError("write your pallas kernel here")


# speedup vs baseline: 29.1641x; 29.1641x over previous
import functools

import jax
import jax.numpy as jnp
from jax import lax
from jax.experimental import pallas as pl
from jax.experimental.pallas import tpu as pltpu
from jax.experimental.pallas import tpu_sc as plsc

N_NODES = 10000
N_PAD = 10016
N_EDGES = 160000
E_PAD = 163840
NUM_THETAS = 16
BUMP_STEPS = 16
NUM_GRAPHS = 32
SCALE = 500.0

NW = 32
EW = E_PAD // NW
CHUNK = 2560
BE = 4096


def _node_body(x_ref, v_ref, b_ref, lin_ref, nh_ref, acc_ref):
    nh = jnp.dot(x_ref[...], v_ref[...], preferred_element_type=jnp.float32)
    nh_ref[...] = nh
    vals = jnp.concatenate([nh] * BUMP_STEPS, axis=1)
    sig = 1.0 / (1.0 + jnp.exp(SCALE * (vals - lin_ref[...])))
    iota_g = lax.broadcasted_iota(jnp.int32, (1, NUM_GRAPHS), 1)
    oh = (b_ref[...] == iota_g).astype(jnp.float32)
    acc_ref[...] = lax.dot_general(
        oh, sig, (((0,), (0,)), ((), ())),
        preferred_element_type=jnp.float32)


@functools.cache
def _build_sc_gather():
    mesh = plsc.VectorSubcoreMesh(core_axis_name="c", subcore_axis_name="s")

    @functools.partial(
        pl.kernel,
        mesh=mesh,
        compiler_params=pltpu.CompilerParams(use_tc_tiling_on_sc=False),
        out_type=[
            jax.ShapeDtypeStruct((E_PAD, 16), jnp.float32),
            jax.ShapeDtypeStruct((E_PAD, 16), jnp.float32),
            jax.ShapeDtypeStruct((E_PAD,), jnp.int32),
        ],
        scratch_types=[
            pltpu.VMEM((CHUNK,), jnp.int32),
            pltpu.VMEM((CHUNK,), jnp.int32),
            pltpu.VMEM((CHUNK, 16), jnp.float32),
            pltpu.VMEM((CHUNK, 16), jnp.float32),
            pltpu.VMEM((CHUNK,), jnp.int32),
            pltpu.SemaphoreType.DMA,
            pltpu.SemaphoreType.DMA,
            pltpu.SemaphoreType.DMA,
        ],
    )
    def _sc_gather(u_hbm, w_hbm, nh_hbm, batch_hbm, ehu_hbm, ehw_hbm, gb_hbm,
                   iu_v, iw_v, ru_v, rw_v, gbv_v, s1, s2, s3):
        wid = lax.axis_index("s") * 2 + lax.axis_index("c")
        base = wid * EW
        for r in range(EW // CHUNK):
            off = base + r * CHUNK
            pltpu.sync_copy(u_hbm.at[pl.ds(off, CHUNK)], iu_v)
            pltpu.sync_copy(w_hbm.at[pl.ds(off, CHUNK)], iw_v)
            cu = pltpu.async_copy(nh_hbm.at[iu_v], ru_v, s1)
            cw = pltpu.async_copy(nh_hbm.at[iw_v], rw_v, s2)
            cg = pltpu.async_copy(batch_hbm.at[iu_v], gbv_v, s3)
            cu.wait()
            cw.wait()
            cg.wait()
            pltpu.sync_copy(ru_v, ehu_hbm.at[pl.ds(off, CHUNK)])
            pltpu.sync_copy(rw_v, ehw_hbm.at[pl.ds(off, CHUNK)])
            pltpu.sync_copy(gbv_v, gb_hbm.at[pl.ds(off, CHUNK)])

    return _sc_gather


def _edge_body(nacc_ref, ehu_ref, ehw_ref, gb_ref, lin_ref, out_ref):
    m = jnp.maximum(ehu_ref[...], ehw_ref[...])
    vals = jnp.concatenate([m] * BUMP_STEPS, axis=1)
    sig = 1.0 / (1.0 + jnp.exp(SCALE * (vals - lin_ref[...])))
    iota_g = lax.broadcasted_iota(jnp.int32, (1, NUM_GRAPHS), 1)
    oh = (gb_ref[...] == iota_g).astype(jnp.float32)
    c = lax.dot_general(oh, sig, (((0,), (0,)), ((), ())),
                        preferred_element_type=jnp.float32)

    @pl.when(pl.program_id(0) == 0)
    def _():
        out_ref[...] = nacc_ref[...] - c

    @pl.when(pl.program_id(0) != 0)
    def _():
        out_ref[...] = out_ref[...] - c


def kernel(x, edge_index, batch, v, lin):
    f32 = jnp.float32
    i32 = jnp.int32
    xp = jnp.concatenate([x, jnp.zeros((N_PAD - N_NODES, x.shape[1]), f32)])
    batchp = jnp.concatenate(
        [batch.astype(i32), jnp.full((N_PAD - N_NODES,), -1, i32)])
    ei_pad = jnp.concatenate(
        [edge_index.astype(i32),
         jnp.full((2, E_PAD - N_EDGES), N_NODES, i32)], axis=1)
    u_flat = ei_pad[0]
    w_flat = ei_pad[1]
    linrep = jnp.repeat(lin.reshape(BUMP_STEPS).astype(f32),
                        NUM_THETAS).reshape(1, BUMP_STEPS * NUM_THETAS)

    nh_ext, nodeacc = pl.pallas_call(
        _node_body,
        out_shape=[
            jax.ShapeDtypeStruct((N_PAD, NUM_THETAS), f32),
            jax.ShapeDtypeStruct((NUM_GRAPHS, BUMP_STEPS * NUM_THETAS), f32),
        ],
    )(xp, v, batchp.reshape(N_PAD, 1), linrep)

    ehu, ehw, gb = _build_sc_gather()(u_flat, w_flat, nh_ext, batchp)

    out = pl.pallas_call(
        _edge_body,
        grid=(E_PAD // BE,),
        in_specs=[
            pl.BlockSpec((NUM_GRAPHS, 256), lambda i: (0, 0)),
            pl.BlockSpec((BE, 16), lambda i: (i, 0)),
            pl.BlockSpec((BE, 16), lambda i: (i, 0)),
            pl.BlockSpec((BE, 1), lambda i: (i, 0)),
            pl.BlockSpec((1, 256), lambda i: (0, 0)),
        ],
        out_specs=pl.BlockSpec((NUM_GRAPHS, 256), lambda i: (0, 0)),
        out_shape=jax.ShapeDtypeStruct((NUM_GRAPHS, 256), f32),
    )(nodeacc, ehu, ehw, gb.reshape(E_PAD, 1), linrep)

    return out.reshape(NUM_GRAPHS, BUMP_STEPS, NUM_THETAS)


# speedup vs baseline: 48.6906x; 1.6695x over previous
import functools

import jax
import jax.numpy as jnp
from jax import lax
from jax.experimental import pallas as pl
from jax.experimental.pallas import tpu as pltpu
from jax.experimental.pallas import tpu_sc as plsc

N_NODES = 10000
N_PAD = 10016
N_EDGES = 160000
E_PAD = 163840
NUM_THETAS = 16
BUMP_STEPS = 16
NUM_GRAPHS = 32
SB = BUMP_STEPS * NUM_THETAS
SCALE = 500.0

NW = 32
EW = E_PAD // NW
CHUNK = 2560
BE = 4096


def _repeat_mat():
    row = lax.broadcasted_iota(jnp.int32, (NUM_THETAS, SB), 0)
    col = lax.broadcasted_iota(jnp.int32, (NUM_THETAS, SB), 1)
    return (col % NUM_THETAS == row).astype(jnp.float32)


def _node_body(x_ref, v_ref, b_ref, lin_ref, nh_ref, acc_ref):
    nh = jnp.dot(x_ref[...], v_ref[...], preferred_element_type=jnp.float32)
    nh_ref[0:N_NODES, :] = nh
    nh_ref[N_NODES:N_PAD, :] = jnp.zeros((N_PAD - N_NODES, NUM_THETAS),
                                         jnp.float32)
    vals = jnp.dot(nh, _repeat_mat(),
                   preferred_element_type=jnp.float32)
    sig = 1.0 / (1.0 + jnp.exp(SCALE * (vals - lin_ref[...])))
    iota_g = lax.broadcasted_iota(jnp.int32, (1, NUM_GRAPHS), 1)
    oh = (b_ref[...] == iota_g).astype(jnp.float32)
    acc_ref[...] = lax.dot_general(
        oh, sig, (((0,), (0,)), ((), ())),
        preferred_element_type=jnp.float32)


@functools.cache
def _build_sc_gather():
    mesh = plsc.VectorSubcoreMesh(core_axis_name="c", subcore_axis_name="s")

    @functools.partial(
        pl.kernel,
        mesh=mesh,
        compiler_params=pltpu.CompilerParams(use_tc_tiling_on_sc=False),
        out_type=[
            jax.ShapeDtypeStruct((E_PAD, 16), jnp.float32),
            jax.ShapeDtypeStruct((E_PAD,), jnp.int32),
        ],
        scratch_types=[
            pltpu.VMEM((CHUNK,), jnp.int32),
            pltpu.VMEM((CHUNK,), jnp.int32),
            pltpu.VMEM((CHUNK, 16), jnp.float32),
            pltpu.VMEM((CHUNK, 16), jnp.float32),
            pltpu.VMEM((CHUNK,), jnp.int32),
            pltpu.SemaphoreType.DMA,
            pltpu.SemaphoreType.DMA,
            pltpu.SemaphoreType.DMA,
        ],
    )
    def _sc_gather(u_hbm, w_hbm, nh_hbm, batch_hbm, eh_hbm, gb_hbm,
                   iu_v, iw_v, ru_v, rw_v, gbv_v, s1, s2, s3):
        wid = lax.axis_index("s") * 2 + lax.axis_index("c")
        base = wid * EW
        for r in range(EW // CHUNK):
            off = base + r * CHUNK
            pltpu.sync_copy(u_hbm.at[pl.ds(off, CHUNK)], iu_v)
            pltpu.sync_copy(w_hbm.at[pl.ds(off, CHUNK)], iw_v)
            cu = pltpu.async_copy(nh_hbm.at[iu_v], ru_v, s1)
            cw = pltpu.async_copy(nh_hbm.at[iw_v], rw_v, s2)
            cg = pltpu.async_copy(batch_hbm.at[iu_v], gbv_v, s3)
            cu.wait()
            cw.wait()
            cg.wait()

            def _mx(i, c):
                ru_v[i] = jnp.maximum(ru_v[i], rw_v[i])
                return c

            lax.fori_loop(0, CHUNK, _mx, 0)
            pltpu.sync_copy(ru_v, eh_hbm.at[pl.ds(off, CHUNK)])
            pltpu.sync_copy(gbv_v, gb_hbm.at[pl.ds(off, CHUNK)])

    return _sc_gather


def _edge_body(nacc_ref, eh_ref, gb_ref, lin_ref, out_ref):
    vals = jnp.dot(eh_ref[...], _repeat_mat(),
                   preferred_element_type=jnp.float32)
    sig = 1.0 / (1.0 + jnp.exp(SCALE * (vals - lin_ref[...])))
    iota_g = lax.broadcasted_iota(jnp.int32, (1, NUM_GRAPHS), 1)
    oh = (gb_ref[...] == iota_g).astype(jnp.float32)
    c = lax.dot_general(oh, sig, (((0,), (0,)), ((), ())),
                        preferred_element_type=jnp.float32)

    @pl.when(pl.program_id(0) == 0)
    def _():
        out_ref[...] = nacc_ref[...] - c

    @pl.when(pl.program_id(0) != 0)
    def _():
        out_ref[...] = out_ref[...] - c


def kernel(x, edge_index, batch, v, lin):
    f32 = jnp.float32
    i32 = jnp.int32
    batchp = jnp.concatenate(
        [batch.astype(i32), jnp.full((N_PAD - N_NODES,), -1, i32)])
    ei_pad = jnp.concatenate(
        [edge_index.astype(i32),
         jnp.full((2, E_PAD - N_EDGES), N_NODES, i32)], axis=1)
    u_flat = ei_pad[0]
    w_flat = ei_pad[1]
    linrep = jnp.repeat(lin.reshape(BUMP_STEPS).astype(f32),
                        NUM_THETAS).reshape(1, SB)

    nh_ext, nodeacc = pl.pallas_call(
        _node_body,
        out_shape=[
            jax.ShapeDtypeStruct((N_PAD, NUM_THETAS), f32),
            jax.ShapeDtypeStruct((NUM_GRAPHS, SB), f32),
        ],
    )(x, v, batch.reshape(N_NODES, 1), linrep)

    eh, gb = _build_sc_gather()(u_flat, w_flat, nh_ext, batchp)

    out = pl.pallas_call(
        _edge_body,
        grid=(E_PAD // BE,),
        in_specs=[
            pl.BlockSpec((NUM_GRAPHS, SB), lambda i: (0, 0)),
            pl.BlockSpec((BE, 16), lambda i: (i, 0)),
            pl.BlockSpec((BE, 1), lambda i: (i, 0)),
            pl.BlockSpec((1, SB), lambda i: (0, 0)),
        ],
        out_specs=pl.BlockSpec((NUM_GRAPHS, SB), lambda i: (0, 0)),
        out_shape=jax.ShapeDtypeStruct((NUM_GRAPHS, SB), f32),
    )(nodeacc, eh, gb.reshape(E_PAD, 1), linrep)

    return out.reshape(NUM_GRAPHS, BUMP_STEPS, NUM_THETAS)


# speedup vs baseline: 57.5506x; 1.1820x over previous
import functools

import jax
import jax.numpy as jnp
from jax import lax
from jax.experimental import pallas as pl
from jax.experimental.pallas import tpu as pltpu
from jax.experimental.pallas import tpu_sc as plsc

N_NODES = 10000
N_PAD = 10016
N_EDGES = 160000
E_PAD = 163840
NUM_THETAS = 16
BUMP_STEPS = 16
NUM_GRAPHS = 32
SB = BUMP_STEPS * NUM_THETAS
SCALE = 500.0

NW = 32
EW = E_PAD // NW
CHUNK = 1280
NROUND = EW // CHUNK
BE = 4096
BEP = BE * 16 // 128
JP = 128 // NUM_THETAS
SBJ = SB * JP


def _repeat_mat():
    row = lax.broadcasted_iota(jnp.int32, (NUM_THETAS, SB), 0)
    col = lax.broadcasted_iota(jnp.int32, (NUM_THETAS, SB), 1)
    return (col % NUM_THETAS == row).astype(jnp.float32)


def _node_body(x_ref, v_ref, b_ref, lin_ref, nh_ref, acc_ref, st_ref):
    nh = jnp.dot(x_ref[...], v_ref[...], preferred_element_type=jnp.float32)
    nh_ref[0:N_NODES, :] = nh
    nh_ref[N_NODES:N_PAD, :] = jnp.zeros((N_PAD - N_NODES, NUM_THETAS),
                                         jnp.float32)
    vals = jnp.dot(nh, _repeat_mat(),
                   preferred_element_type=jnp.float32)
    sig = 1.0 / (1.0 + jnp.exp(SCALE * (vals - lin_ref[...])))
    iota_g = lax.broadcasted_iota(jnp.int32, (1, NUM_GRAPHS), 1)
    oh = (b_ref[...] == iota_g).astype(jnp.float32)
    acc_ref[...] = lax.dot_general(
        oh, sig, (((0,), (0,)), ((), ())),
        preferred_element_type=jnp.float32)
    le = (b_ref[...] <= iota_g).astype(jnp.float32)
    ends = jnp.sum(le, axis=0, keepdims=True)
    counts = jnp.sum(oh, axis=0, keepdims=True)
    st_ref[...] = jnp.concatenate([ends - counts, ends], axis=0)


@functools.cache
def _build_sc_gather():
    mesh = plsc.VectorSubcoreMesh(core_axis_name="c", subcore_axis_name="s")

    @functools.partial(
        pl.kernel,
        mesh=mesh,
        compiler_params=pltpu.CompilerParams(use_tc_tiling_on_sc=False),
        out_type=jax.ShapeDtypeStruct((E_PAD, 16), jnp.float32),
        scratch_types=[
            pltpu.VMEM((2, CHUNK), jnp.int32),
            pltpu.VMEM((2, CHUNK), jnp.int32),
            pltpu.VMEM((2, CHUNK, 16), jnp.float32),
            pltpu.VMEM((2, CHUNK, 16), jnp.float32),
            pltpu.SemaphoreType.DMA,
            pltpu.SemaphoreType.DMA,
            pltpu.SemaphoreType.DMA,
            pltpu.SemaphoreType.DMA,
        ],
    )
    def _sc_gather(u_hbm, w_hbm, nh_hbm, eh_hbm,
                   iu_v, iw_v, ru_v, rw_v, su0, sw0, su1, sw1):
        wid = lax.axis_index("s") * 2 + lax.axis_index("c")
        base = wid * EW
        sems = ((su0, sw0), (su1, sw1))

        def fire(r, slot):
            off = base + r * CHUNK
            pltpu.sync_copy(u_hbm.at[pl.ds(off, CHUNK)], iu_v.at[slot])
            pltpu.sync_copy(w_hbm.at[pl.ds(off, CHUNK)], iw_v.at[slot])
            cu = pltpu.async_copy(nh_hbm.at[iu_v.at[slot]], ru_v.at[slot],
                                  sems[slot][0])
            cw = pltpu.async_copy(nh_hbm.at[iw_v.at[slot]], rw_v.at[slot],
                                  sems[slot][1])
            return cu, cw

        pend = fire(0, 0)
        for r in range(NROUND):
            slot = r % 2
            cu, cw = pend
            if r + 1 < NROUND:
                nxt = fire(r + 1, (r + 1) % 2)
            cu.wait()
            cw.wait()

            def _mx(i, c, _slot=slot):
                for k in range(4):
                    j = i * 4 + k
                    ru_v[_slot, j] = jnp.maximum(ru_v[_slot, j],
                                                 rw_v[_slot, j])
                return c

            lax.fori_loop(0, CHUNK // 4, _mx, 0)
            off = base + r * CHUNK
            pltpu.sync_copy(ru_v.at[slot], eh_hbm.at[pl.ds(off, CHUNK)])
            if r + 1 < NROUND:
                pend = nxt

    return _sc_gather


def _edge_body(nacc_ref, eh_ref, u_ref, st_ref, lin_ref, out_ref):
    f32 = jnp.float32
    vals = jnp.dot(eh_ref[...], _repeat_mat(),
                   preferred_element_type=f32)
    sig = 1.0 / (1.0 + jnp.exp(SCALE * (vals - lin_ref[...])))
    uf = u_ref[...].astype(f32)
    oh = ((uf >= st_ref[0:1, :] - 0.5) & (uf < st_ref[1:2, :] - 0.5)
          ).astype(f32)
    c = lax.dot_general(oh, sig, (((0,), (0,)), ((), ())),
                        preferred_element_type=f32)

    @pl.when(pl.program_id(0) == 0)
    def _():
        out_ref[...] = nacc_ref[...] - c

    @pl.when(pl.program_id(0) != 0)
    def _():
        out_ref[...] = out_ref[...] - c


def kernel(x, edge_index, batch, v, lin):
    f32 = jnp.float32
    i32 = jnp.int32
    ei_pad = jnp.concatenate(
        [edge_index.astype(i32),
         jnp.full((2, E_PAD - N_EDGES), N_NODES, i32)], axis=1)
    u_flat = ei_pad[0]
    w_flat = ei_pad[1]
    linrep = jnp.repeat(lin.reshape(BUMP_STEPS).astype(f32),
                        NUM_THETAS).reshape(1, SB)

    nh_ext, nodeacc, st = pl.pallas_call(
        _node_body,
        out_shape=[
            jax.ShapeDtypeStruct((N_PAD, NUM_THETAS), f32),
            jax.ShapeDtypeStruct((NUM_GRAPHS, SB), f32),
            jax.ShapeDtypeStruct((2, NUM_GRAPHS), f32),
        ],
    )(x, v, batch.reshape(N_NODES, 1), linrep)

    eh = _build_sc_gather()(u_flat, w_flat, nh_ext)

    out = pl.pallas_call(
        _edge_body,
        grid=(E_PAD // BE,),
        in_specs=[
            pl.BlockSpec((NUM_GRAPHS, SB), lambda i: (0, 0)),
            pl.BlockSpec((BE, 16), lambda i: (i, 0)),
            pl.BlockSpec((BE, 1), lambda i: (i, 0)),
            pl.BlockSpec((2, NUM_GRAPHS), lambda i: (0, 0)),
            pl.BlockSpec((1, SB), lambda i: (0, 0)),
        ],
        out_specs=pl.BlockSpec((NUM_GRAPHS, SB), lambda i: (0, 0)),
        out_shape=jax.ShapeDtypeStruct((NUM_GRAPHS, SB), f32),
    )(nodeacc, eh, u_flat.reshape(E_PAD, 1), st, linrep)

    return out.reshape(NUM_GRAPHS, BUMP_STEPS, NUM_THETAS)


# speedup vs baseline: 74.4742x; 1.2941x over previous
import functools

import jax
import jax.numpy as jnp
from jax import lax
from jax.experimental import pallas as pl
from jax.experimental.pallas import tpu as pltpu
from jax.experimental.pallas import tpu_sc as plsc

N_NODES = 10000
N_PAD = 10016
N_EDGES = 160000
E_PAD = 163840
NUM_THETAS = 16
BUMP_STEPS = 16
NUM_GRAPHS = 32
SB = BUMP_STEPS * NUM_THETAS
SCALE = 500.0

NW = 32
EW = E_PAD // NW
CHUNK = 1280
NROUND = EW // CHUNK
BE = 4096
BEP = BE * 16 // 128
JP = 128 // NUM_THETAS
SBJ = SB * JP


def _repeat_mat():
    row = lax.broadcasted_iota(jnp.int32, (NUM_THETAS, SB), 0)
    col = lax.broadcasted_iota(jnp.int32, (NUM_THETAS, SB), 1)
    return (col % NUM_THETAS == row).astype(jnp.float32)


def _node_body(x_ref, v_ref, b_ref, lin_ref, nh_ref, acc_ref, st_ref):
    nh = jnp.dot(x_ref[...], v_ref[...], preferred_element_type=jnp.float32)
    nh_ref[0:N_NODES, :] = nh
    nh_ref[N_NODES:N_PAD, :] = jnp.zeros((N_PAD - N_NODES, NUM_THETAS),
                                         jnp.float32)
    vals = jnp.dot(nh, _repeat_mat(),
                   preferred_element_type=jnp.float32)
    sig = 1.0 / (1.0 + jnp.exp(SCALE * (vals - lin_ref[...])))
    iota_g = lax.broadcasted_iota(jnp.int32, (1, NUM_GRAPHS), 1)
    oh = (b_ref[...] == iota_g).astype(jnp.float32)
    acc_ref[...] = lax.dot_general(
        oh, sig, (((0,), (0,)), ((), ())),
        preferred_element_type=jnp.float32)
    le = (b_ref[...] <= iota_g).astype(jnp.float32)
    ends = jnp.sum(le, axis=0, keepdims=True)
    counts = jnp.sum(oh, axis=0, keepdims=True)
    st_ref[...] = jnp.concatenate([ends - counts, ends], axis=0)


@functools.cache
def _build_sc_gather():
    mesh = plsc.VectorSubcoreMesh(core_axis_name="c", subcore_axis_name="s")

    @functools.partial(
        pl.kernel,
        mesh=mesh,
        compiler_params=pltpu.CompilerParams(use_tc_tiling_on_sc=False),
        out_type=jax.ShapeDtypeStruct((E_PAD // JP, 128), jnp.float32),
        scratch_types=[
            pltpu.VMEM((2, CHUNK), jnp.int32),
            pltpu.VMEM((2, CHUNK), jnp.int32),
            pltpu.VMEM((2, CHUNK, 16), jnp.float32),
            pltpu.VMEM((2, CHUNK, 16), jnp.float32),
            pltpu.VMEM((CHUNK // JP, 128), jnp.float32),
            pltpu.SemaphoreType.DMA,
            pltpu.SemaphoreType.DMA,
            pltpu.SemaphoreType.DMA,
            pltpu.SemaphoreType.DMA,
        ],
    )
    def _sc_gather(u_hbm, w_hbm, nh_hbm, eh_hbm,
                   iu_v, iw_v, ru_v, rw_v, fl_v, su0, sw0, su1, sw1):
        wid = lax.axis_index("s") * 2 + lax.axis_index("c")
        base = wid * EW
        sems = ((su0, sw0), (su1, sw1))

        def fire(r, slot):
            off = base + r * CHUNK
            pltpu.sync_copy(u_hbm.at[pl.ds(off, CHUNK)], iu_v.at[slot])
            pltpu.sync_copy(w_hbm.at[pl.ds(off, CHUNK)], iw_v.at[slot])
            cu = pltpu.async_copy(nh_hbm.at[iu_v.at[slot]], ru_v.at[slot],
                                  sems[slot][0])
            cw = pltpu.async_copy(nh_hbm.at[iw_v.at[slot]], rw_v.at[slot],
                                  sems[slot][1])
            return cu, cw

        pend = fire(0, 0)
        for r in range(NROUND):
            slot = r % 2
            cu, cw = pend
            if r + 1 < NROUND:
                nxt = fire(r + 1, (r + 1) % 2)
            cu.wait()
            cw.wait()

            def _mx(i, c, _slot=slot):
                for k in range(JP):
                    j = i * JP + k
                    fl_v[i, pl.ds(k * NUM_THETAS, NUM_THETAS)] = jnp.maximum(
                        ru_v[_slot, j], rw_v[_slot, j])
                return c

            lax.fori_loop(0, CHUNK // JP, _mx, 0)
            row = (base + r * CHUNK) // JP
            pltpu.sync_copy(fl_v, eh_hbm.at[pl.ds(row, CHUNK // JP)])
            if r + 1 < NROUND:
                pend = nxt

    return _sc_gather


def _edge_body_r4(nacc_ref, eh_ref, u_ref, st_ref, lin_ref, out_ref):
    f32 = jnp.float32
    vals = jnp.dot(eh_ref[...], _repeat_mat(),
                   preferred_element_type=f32)
    sig = 1.0 / (1.0 + jnp.exp(SCALE * (vals - lin_ref[...])))
    uf = u_ref[...].astype(f32)
    oh = ((uf >= st_ref[0:1, :] - 0.5) & (uf < st_ref[1:2, :] - 0.5)
          ).astype(f32)
    c = lax.dot_general(oh, sig, (((0,), (0,)), ((), ())),
                        preferred_element_type=f32)

    @pl.when(pl.program_id(0) == 0)
    def _():
        out_ref[...] = nacc_ref[...] - c

    @pl.when(pl.program_id(0) != 0)
    def _():
        out_ref[...] = out_ref[...] - c


def _edge_body(nacc_ref, eh_ref, u8_ref, stp_ref, lin16_ref, out_ref,
               acc_ref):
    i32 = jnp.int32
    f32 = jnp.float32
    m2 = eh_ref[...]
    pieces = [
        1.0 / (1.0 + jnp.exp(SCALE * (m2 - lin16_ref[0, s])))
        for s in range(BUMP_STEPS)
    ]
    sigP = jnp.concatenate(pieces, axis=1)
    u8f = u8_ref[...]
    st0 = stp_ref[0:1, 0:NUM_GRAPHS]
    st1 = stp_ref[1:2, 0:NUM_GRAPHS]
    ohs = []
    for j in range(JP):
        uj = u8f[:, j:j + 1]
        ohs.append(((uj >= st0 - 0.5) & (uj < st1 - 0.5)).astype(f32))
    ohp = jnp.concatenate(ohs, axis=1)
    c = lax.dot_general(ohp, sigP, (((0,), (0,)), ((), ())),
                        preferred_element_type=f32)

    @pl.when(pl.program_id(0) == 0)
    def _():
        acc_ref[...] = c

    @pl.when(pl.program_id(0) != 0)
    def _():
        acc_ref[...] = acc_ref[...] + c

    @pl.when(pl.program_id(0) == pl.num_programs(0) - 1)
    def _():
        cols = []
        for s in range(BUMP_STEPS):
            blk = acc_ref[0:NUM_GRAPHS,
                          s * 128:s * 128 + NUM_THETAS]
            for j in range(1, JP):
                blk = blk + acc_ref[j * NUM_GRAPHS:(j + 1) * NUM_GRAPHS,
                                    s * 128 + j * NUM_THETAS:
                                    s * 128 + (j + 1) * NUM_THETAS]
            cols.append(blk)
        out_ref[...] = nacc_ref[...] - jnp.concatenate(cols, axis=1)


def kernel(x, edge_index, batch, v, lin):
    f32 = jnp.float32
    i32 = jnp.int32
    ei_pad = jnp.concatenate(
        [edge_index.astype(i32),
         jnp.full((2, E_PAD - N_EDGES), N_NODES, i32)], axis=1)
    u_flat = ei_pad[0]
    w_flat = ei_pad[1]
    linrep = jnp.repeat(lin.reshape(BUMP_STEPS).astype(f32),
                        NUM_THETAS).reshape(1, SB)
    lin_p = jnp.concatenate([linrep] * JP, axis=1)

    nh_ext, nodeacc, st = pl.pallas_call(
        _node_body,
        out_shape=[
            jax.ShapeDtypeStruct((N_PAD, NUM_THETAS), f32),
            jax.ShapeDtypeStruct((NUM_GRAPHS, SB), f32),
            jax.ShapeDtypeStruct((2, NUM_GRAPHS), f32),
        ],
    )(x, v, batch.reshape(N_NODES, 1), linrep)

    eh = _build_sc_gather()(u_flat, w_flat, nh_ext)

    u8 = jnp.concatenate(
        [u_flat.astype(f32).reshape(E_PAD // JP, JP),
         jnp.zeros((E_PAD // JP, 128 - JP), f32)], axis=1)
    stp = jnp.tile(st, (1, JP))
    lin16 = lin.reshape(1, BUMP_STEPS).astype(f32)
    out = pl.pallas_call(
        _edge_body,
        grid=(E_PAD // BE,),
        in_specs=[
            pl.BlockSpec((NUM_GRAPHS, SB), lambda i: (0, 0)),
            pl.BlockSpec((BEP, 128), lambda i: (i, 0)),
            pl.BlockSpec((BE // JP, 128), lambda i: (i, 0)),
            pl.BlockSpec((2, SB), lambda i: (0, 0)),
            pl.BlockSpec((1, BUMP_STEPS), lambda i: (0, 0)),
        ],
        out_specs=pl.BlockSpec((NUM_GRAPHS, SB), lambda i: (0, 0)),
        out_shape=jax.ShapeDtypeStruct((NUM_GRAPHS, SB), f32),
        scratch_shapes=[pltpu.VMEM((JP * NUM_GRAPHS, SBJ), f32)],
    )(nodeacc, eh, u8, stp, lin16)

    return out.reshape(NUM_GRAPHS, BUMP_STEPS, NUM_THETAS)


# speedup vs baseline: 75.0826x; 1.0082x over previous
import functools

import jax
import jax.numpy as jnp
from jax import lax
from jax.experimental import pallas as pl
from jax.experimental.pallas import tpu as pltpu
from jax.experimental.pallas import tpu_sc as plsc

N_NODES = 10000
N_PAD = 10016
N_EDGES = 160000
E_PAD = 163840
NUM_THETAS = 16
BUMP_STEPS = 16
NUM_GRAPHS = 32
SB = BUMP_STEPS * NUM_THETAS
SCALE = 500.0

NW = 32
EW = E_PAD // NW
CHUNK = 1280
NROUND = EW // CHUNK
BE = 4096
BEP = BE * 16 // 128
JP = 128 // NUM_THETAS
SBJ = SB * JP


def _repeat_mat():
    row = lax.broadcasted_iota(jnp.int32, (NUM_THETAS, SB), 0)
    col = lax.broadcasted_iota(jnp.int32, (NUM_THETAS, SB), 1)
    return (col % NUM_THETAS == row).astype(jnp.float32)


def _node_body(x_ref, v_ref, b_ref, lin_ref, nh_ref, acc_ref, st_ref):
    nh = jnp.dot(x_ref[...], v_ref[...], preferred_element_type=jnp.float32)
    nh_ref[0:N_NODES, :] = nh
    nh_ref[N_NODES:N_PAD, :] = jnp.zeros((N_PAD - N_NODES, NUM_THETAS),
                                         jnp.float32)
    vals = jnp.dot(nh, _repeat_mat(),
                   preferred_element_type=jnp.float32)
    sig = 1.0 / (1.0 + jnp.exp(SCALE * (vals - lin_ref[...])))
    iota_g = lax.broadcasted_iota(jnp.int32, (1, NUM_GRAPHS), 1)
    oh = (b_ref[...] == iota_g).astype(jnp.float32)
    acc_ref[...] = lax.dot_general(
        oh, sig, (((0,), (0,)), ((), ())),
        preferred_element_type=jnp.float32)
    le = (b_ref[...] <= iota_g).astype(jnp.float32)
    ends = jnp.sum(le, axis=0, keepdims=True)
    counts = jnp.sum(oh, axis=0, keepdims=True)
    st_ref[...] = jnp.concatenate([ends - counts, ends], axis=0)


@functools.cache
def _build_sc_gather():
    mesh = plsc.VectorSubcoreMesh(core_axis_name="c", subcore_axis_name="s")

    @functools.partial(
        pl.kernel,
        mesh=mesh,
        compiler_params=pltpu.CompilerParams(use_tc_tiling_on_sc=False),
        out_type=jax.ShapeDtypeStruct((E_PAD // JP, 128), jnp.float32),
        scratch_types=[
            pltpu.VMEM((2, CHUNK), jnp.int32),
            pltpu.VMEM((2, CHUNK), jnp.int32),
            pltpu.VMEM((2, CHUNK, 16), jnp.float32),
            pltpu.VMEM((2, CHUNK, 16), jnp.float32),
            pltpu.VMEM((CHUNK // JP, 128), jnp.float32),
            pltpu.SemaphoreType.DMA,
            pltpu.SemaphoreType.DMA,
            pltpu.SemaphoreType.DMA,
            pltpu.SemaphoreType.DMA,
        ],
    )
    def _sc_gather(u_hbm, w_hbm, nh_hbm, eh_hbm,
                   iu_v, iw_v, ru_v, rw_v, fl_v, su0, sw0, su1, sw1):
        cid = lax.axis_index("c")
        sid = lax.axis_index("s")
        sems = ((su0, sw0), (su1, sw1))

        def run(base, nround):
            def fire(r, slot):
                off = base + r * CHUNK
                pltpu.sync_copy(u_hbm.at[pl.ds(off, CHUNK)], iu_v.at[slot])
                pltpu.sync_copy(w_hbm.at[pl.ds(off, CHUNK)], iw_v.at[slot])
                cu = pltpu.async_copy(nh_hbm.at[iu_v.at[slot]],
                                      ru_v.at[slot], sems[slot][0])
                cw = pltpu.async_copy(nh_hbm.at[iw_v.at[slot]],
                                      rw_v.at[slot], sems[slot][1])
                return cu, cw

            pend = fire(0, 0)
            for r in range(nround):
                slot = r % 2
                cu, cw = pend
                if r + 1 < nround:
                    nxt = fire(r + 1, (r + 1) % 2)
                cu.wait()
                cw.wait()

                def _mx(i, c, _slot=slot):
                    for k in range(JP):
                        j = i * JP + k
                        fl_v[i, pl.ds(k * NUM_THETAS, NUM_THETAS)] = (
                            jnp.maximum(ru_v[_slot, j], rw_v[_slot, j]))
                    return c

                lax.fori_loop(0, CHUNK // JP, _mx, 0)
                row = (base + r * CHUNK) // JP
                pltpu.sync_copy(fl_v, eh_hbm.at[pl.ds(row, CHUNK // JP)])
                if r + 1 < nround:
                    pend = nxt

        pair = sid * (2 * EW)

        @pl.when(cid == 0)
        def _():
            run(pair, 2 * NROUND - 2)

        @pl.when(cid != 0)
        def _():
            run(pair + (2 * NROUND - 2) * CHUNK, 2)

    return _sc_gather


def _edge_body_r4(nacc_ref, eh_ref, u_ref, st_ref, lin_ref, out_ref):
    f32 = jnp.float32
    vals = jnp.dot(eh_ref[...], _repeat_mat(),
                   preferred_element_type=f32)
    sig = 1.0 / (1.0 + jnp.exp(SCALE * (vals - lin_ref[...])))
    uf = u_ref[...].astype(f32)
    oh = ((uf >= st_ref[0:1, :] - 0.5) & (uf < st_ref[1:2, :] - 0.5)
          ).astype(f32)
    c = lax.dot_general(oh, sig, (((0,), (0,)), ((), ())),
                        preferred_element_type=f32)

    @pl.when(pl.program_id(0) == 0)
    def _():
        out_ref[...] = nacc_ref[...] - c

    @pl.when(pl.program_id(0) != 0)
    def _():
        out_ref[...] = out_ref[...] - c


def _edge_body(nacc_ref, eh_ref, u8_ref, stp_ref, lin16_ref, out_ref,
               acc_ref):
    i32 = jnp.int32
    f32 = jnp.float32
    m2 = eh_ref[...]
    pieces = [
        1.0 / (1.0 + jnp.exp(SCALE * (m2 - lin16_ref[0, s])))
        for s in range(BUMP_STEPS)
    ]
    sigP = jnp.concatenate(pieces, axis=1)
    u8f = u8_ref[...]
    st0 = stp_ref[0:1, 0:NUM_GRAPHS]
    st1 = stp_ref[1:2, 0:NUM_GRAPHS]
    ohs = []
    for j in range(JP):
        uj = u8f[:, j:j + 1]
        ohs.append(((uj >= st0 - 0.5) & (uj < st1 - 0.5)).astype(f32))
    ohp = jnp.concatenate(ohs, axis=1)
    c = lax.dot_general(ohp.astype(jnp.bfloat16),
                        sigP.astype(jnp.bfloat16),
                        (((0,), (0,)), ((), ())),
                        preferred_element_type=f32)

    @pl.when(pl.program_id(0) == 0)
    def _():
        acc_ref[...] = c

    @pl.when(pl.program_id(0) != 0)
    def _():
        acc_ref[...] = acc_ref[...] + c

    @pl.when(pl.program_id(0) == pl.num_programs(0) - 1)
    def _():
        cols = []
        for s in range(BUMP_STEPS):
            blk = acc_ref[0:NUM_GRAPHS,
                          s * 128:s * 128 + NUM_THETAS]
            for j in range(1, JP):
                blk = blk + acc_ref[j * NUM_GRAPHS:(j + 1) * NUM_GRAPHS,
                                    s * 128 + j * NUM_THETAS:
                                    s * 128 + (j + 1) * NUM_THETAS]
            cols.append(blk)
        out_ref[...] = nacc_ref[...] - jnp.concatenate(cols, axis=1)


def kernel(x, edge_index, batch, v, lin):
    f32 = jnp.float32
    i32 = jnp.int32
    ei_pad = jnp.concatenate(
        [edge_index.astype(i32),
         jnp.full((2, E_PAD - N_EDGES), N_NODES, i32)], axis=1)
    u_flat = ei_pad[0]
    w_flat = ei_pad[1]
    linrep = jnp.repeat(lin.reshape(BUMP_STEPS).astype(f32),
                        NUM_THETAS).reshape(1, SB)
    lin_p = jnp.concatenate([linrep] * JP, axis=1)

    nh_ext, nodeacc, st = pl.pallas_call(
        _node_body,
        out_shape=[
            jax.ShapeDtypeStruct((N_PAD, NUM_THETAS), f32),
            jax.ShapeDtypeStruct((NUM_GRAPHS, SB), f32),
            jax.ShapeDtypeStruct((2, NUM_GRAPHS), f32),
        ],
    )(x, v, batch.reshape(N_NODES, 1), linrep)

    eh = _build_sc_gather()(u_flat, w_flat, nh_ext)

    u8 = jnp.concatenate(
        [u_flat.astype(f32).reshape(E_PAD // JP, JP),
         jnp.zeros((E_PAD // JP, 128 - JP), f32)], axis=1)
    stp = jnp.tile(st, (1, JP))
    lin16 = lin.reshape(1, BUMP_STEPS).astype(f32)
    out = pl.pallas_call(
        _edge_body,
        grid=(E_PAD // BE,),
        in_specs=[
            pl.BlockSpec((NUM_GRAPHS, SB), lambda i: (0, 0)),
            pl.BlockSpec((BEP, 128), lambda i: (i, 0)),
            pl.BlockSpec((BE // JP, 128), lambda i: (i, 0)),
            pl.BlockSpec((2, SB), lambda i: (0, 0)),
            pl.BlockSpec((1, BUMP_STEPS), lambda i: (0, 0)),
        ],
        out_specs=pl.BlockSpec((NUM_GRAPHS, SB), lambda i: (0, 0)),
        out_shape=jax.ShapeDtypeStruct((NUM_GRAPHS, SB), f32),
        scratch_shapes=[pltpu.VMEM((JP * NUM_GRAPHS, SBJ), f32)],
    )(nodeacc, eh, u8, stp, lin16)

    return out.reshape(NUM_GRAPHS, BUMP_STEPS, NUM_THETAS)


# speedup vs baseline: 85.9957x; 1.1453x over previous
import functools

import jax
import jax.numpy as jnp
from jax import lax
from jax.experimental import pallas as pl
from jax.experimental.pallas import tpu as pltpu
from jax.experimental.pallas import tpu_sc as plsc

N_NODES = 10000
N_PAD = 10016
N_EDGES = 160000
E_PAD = 163840
NUM_THETAS = 16
BUMP_STEPS = 16
NUM_GRAPHS = 32
SB = BUMP_STEPS * NUM_THETAS
SCALE = 500.0

NW = 32
EW = E_PAD // NW
CHUNK = 1280
NROUND = EW // CHUNK
BE = 8192
BEP = BE * 16 // 128
JP = 128 // NUM_THETAS
SBJ = SB * JP


def _repeat_mat():
    row = lax.broadcasted_iota(jnp.int32, (NUM_THETAS, SB), 0)
    col = lax.broadcasted_iota(jnp.int32, (NUM_THETAS, SB), 1)
    return (col % NUM_THETAS == row).astype(jnp.float32)


def _node_body(x_ref, v_ref, b_ref, lin_ref, nh_ref, acc_ref, st_ref):
    nh = jnp.dot(x_ref[...], v_ref[...], preferred_element_type=jnp.float32)
    nh_ref[0:N_NODES, :] = nh
    nh_ref[N_NODES:N_PAD, :] = jnp.zeros((N_PAD - N_NODES, NUM_THETAS),
                                         jnp.float32)
    vals = jnp.dot(nh, _repeat_mat(),
                   preferred_element_type=jnp.float32)
    sig = 1.0 / (1.0 + jnp.exp(SCALE * (vals - lin_ref[...])))
    iota_g = lax.broadcasted_iota(jnp.int32, (1, NUM_GRAPHS), 1)
    oh = (b_ref[...] == iota_g).astype(jnp.float32)
    acc_ref[...] = lax.dot_general(
        oh, sig, (((0,), (0,)), ((), ())),
        preferred_element_type=jnp.float32)
    le = (b_ref[...] <= iota_g).astype(jnp.float32)
    ends = jnp.sum(le, axis=0, keepdims=True)
    counts = jnp.sum(oh, axis=0, keepdims=True)
    st_ref[...] = jnp.concatenate([ends - counts, ends], axis=0)


@functools.cache
def _build_sc_gather():
    mesh = plsc.VectorSubcoreMesh(core_axis_name="c", subcore_axis_name="s")

    @functools.partial(
        pl.kernel,
        mesh=mesh,
        compiler_params=pltpu.CompilerParams(use_tc_tiling_on_sc=False),
        out_type=jax.ShapeDtypeStruct((E_PAD // JP, 128), jnp.float32),
        scratch_types=[
            pltpu.VMEM((2, CHUNK), jnp.int32),
            pltpu.VMEM((2, CHUNK), jnp.int32),
            pltpu.VMEM((2, CHUNK, 16), jnp.float32),
            pltpu.VMEM((2, CHUNK, 16), jnp.float32),
            pltpu.VMEM((CHUNK // JP, 128), jnp.float32),
            pltpu.SemaphoreType.DMA,
            pltpu.SemaphoreType.DMA,
            pltpu.SemaphoreType.DMA,
            pltpu.SemaphoreType.DMA,
        ],
    )
    def _sc_gather(u_hbm, w_hbm, nh_hbm, eh_hbm,
                   iu_v, iw_v, ru_v, rw_v, fl_v, su0, sw0, su1, sw1):
        cid = lax.axis_index("c")
        sid = lax.axis_index("s")
        sems = ((su0, sw0), (su1, sw1))

        def run(base, nround):
            def fire(r, slot):
                off = base + r * CHUNK
                pltpu.sync_copy(u_hbm.at[pl.ds(off, CHUNK)], iu_v.at[slot])
                pltpu.sync_copy(w_hbm.at[pl.ds(off, CHUNK)], iw_v.at[slot])
                cu = pltpu.async_copy(nh_hbm.at[iu_v.at[slot]],
                                      ru_v.at[slot], sems[slot][0])
                cw = pltpu.async_copy(nh_hbm.at[iw_v.at[slot]],
                                      rw_v.at[slot], sems[slot][1])
                return cu, cw

            pend = fire(0, 0)
            for r in range(nround):
                slot = r % 2
                cu, cw = pend
                if r + 1 < nround:
                    nxt = fire(r + 1, (r + 1) % 2)
                cu.wait()
                cw.wait()

                def _mx(i, c, _slot=slot):
                    for k in range(JP):
                        j = i * JP + k
                        fl_v[i, pl.ds(k * NUM_THETAS, NUM_THETAS)] = (
                            jnp.maximum(ru_v[_slot, j], rw_v[_slot, j]))
                    return c

                lax.fori_loop(0, CHUNK // JP, _mx, 0)
                row = (base + r * CHUNK) // JP
                pltpu.sync_copy(fl_v, eh_hbm.at[pl.ds(row, CHUNK // JP)])
                if r + 1 < nround:
                    pend = nxt

        pair = sid * (2 * EW)

        @pl.when(cid == 0)
        def _():
            run(pair, 2 * NROUND - 2)

        @pl.when(cid != 0)
        def _():
            run(pair + (2 * NROUND - 2) * CHUNK, 2)

    return _sc_gather


def _edge_body_r4(nacc_ref, eh_ref, u_ref, st_ref, lin_ref, out_ref):
    f32 = jnp.float32
    vals = jnp.dot(eh_ref[...], _repeat_mat(),
                   preferred_element_type=f32)
    sig = 1.0 / (1.0 + jnp.exp(SCALE * (vals - lin_ref[...])))
    uf = u_ref[...].astype(f32)
    oh = ((uf >= st_ref[0:1, :] - 0.5) & (uf < st_ref[1:2, :] - 0.5)
          ).astype(f32)
    c = lax.dot_general(oh, sig, (((0,), (0,)), ((), ())),
                        preferred_element_type=f32)

    @pl.when(pl.program_id(0) == 0)
    def _():
        out_ref[...] = nacc_ref[...] - c

    @pl.when(pl.program_id(0) != 0)
    def _():
        out_ref[...] = out_ref[...] - c


def _edge_body(nacc_ref, eh_ref, u8_ref, stp_ref, lin16_ref, out_ref,
               acc_ref):
    i32 = jnp.int32
    f32 = jnp.float32
    m2 = eh_ref[...]
    pieces = [
        1.0 / (1.0 + jnp.exp(SCALE * (m2 - lin16_ref[0, s])))
        for s in range(BUMP_STEPS)
    ]
    sigP = jnp.concatenate(pieces, axis=1)
    u8f = u8_ref[...]
    st0 = stp_ref[0:1, 0:NUM_GRAPHS]
    st1 = stp_ref[1:2, 0:NUM_GRAPHS]
    ohs = []
    for j in range(JP):
        uj = u8f[:, j:j + 1]
        ohs.append(((uj >= st0 - 0.5) & (uj < st1 - 0.5)).astype(f32))
    ohp = jnp.concatenate(ohs, axis=1)
    c = lax.dot_general(ohp, sigP, (((0,), (0,)), ((), ())),
                        preferred_element_type=f32)

    @pl.when(pl.program_id(0) == 0)
    def _():
        acc_ref[...] = c

    @pl.when(pl.program_id(0) != 0)
    def _():
        acc_ref[...] = acc_ref[...] + c

    @pl.when(pl.program_id(0) == pl.num_programs(0) - 1)
    def _():
        cols = []
        for s in range(BUMP_STEPS):
            blk = acc_ref[0:NUM_GRAPHS,
                          s * 128:s * 128 + NUM_THETAS]
            for j in range(1, JP):
                blk = blk + acc_ref[j * NUM_GRAPHS:(j + 1) * NUM_GRAPHS,
                                    s * 128 + j * NUM_THETAS:
                                    s * 128 + (j + 1) * NUM_THETAS]
            cols.append(blk)
        out_ref[...] = nacc_ref[...] - jnp.concatenate(cols, axis=1)


def kernel(x, edge_index, batch, v, lin):
    f32 = jnp.float32
    i32 = jnp.int32
    ei_pad = jnp.concatenate(
        [edge_index.astype(i32),
         jnp.full((2, E_PAD - N_EDGES), N_NODES, i32)], axis=1)
    u_flat = ei_pad[0]
    w_flat = ei_pad[1]
    linrep = jnp.repeat(lin.reshape(BUMP_STEPS).astype(f32),
                        NUM_THETAS).reshape(1, SB)
    lin_p = jnp.concatenate([linrep] * JP, axis=1)

    nh_ext, nodeacc, st = pl.pallas_call(
        _node_body,
        out_shape=[
            jax.ShapeDtypeStruct((N_PAD, NUM_THETAS), f32),
            jax.ShapeDtypeStruct((NUM_GRAPHS, SB), f32),
            jax.ShapeDtypeStruct((2, NUM_GRAPHS), f32),
        ],
    )(x, v, batch.reshape(N_NODES, 1), linrep)

    eh = _build_sc_gather()(u_flat, w_flat, nh_ext)

    u8 = u_flat.astype(f32).reshape(E_PAD // JP, JP)
    stp = jnp.tile(st, (1, JP))
    lin16 = lin.reshape(1, BUMP_STEPS).astype(f32)
    out = pl.pallas_call(
        _edge_body,
        grid=(E_PAD // BE,),
        in_specs=[
            pl.BlockSpec((NUM_GRAPHS, SB), lambda i: (0, 0)),
            pl.BlockSpec((BEP, 128), lambda i: (i, 0)),
            pl.BlockSpec((BE // JP, JP), lambda i: (i, 0)),
            pl.BlockSpec((2, SB), lambda i: (0, 0)),
            pl.BlockSpec((1, BUMP_STEPS), lambda i: (0, 0)),
        ],
        out_specs=pl.BlockSpec((NUM_GRAPHS, SB), lambda i: (0, 0)),
        out_shape=jax.ShapeDtypeStruct((NUM_GRAPHS, SB), f32),
        scratch_shapes=[pltpu.VMEM((JP * NUM_GRAPHS, SBJ), f32)],
    )(nodeacc, eh, u8, stp, lin16)

    return out.reshape(NUM_GRAPHS, BUMP_STEPS, NUM_THETAS)


# speedup vs baseline: 86.1284x; 1.0015x over previous
import functools

import jax
import jax.numpy as jnp
from jax import lax
from jax.experimental import pallas as pl
from jax.experimental.pallas import tpu as pltpu
from jax.experimental.pallas import tpu_sc as plsc

N_NODES = 10000
N_PAD = 10016
N_EDGES = 160000
E_PAD = 163840
NUM_THETAS = 16
BUMP_STEPS = 16
NUM_GRAPHS = 32
SB = BUMP_STEPS * NUM_THETAS
SCALE = 500.0

NW = 32
EW = E_PAD // NW
CHUNK = 1280
NROUND = EW // CHUNK
BE = 8192
BEP = BE * 16 // 128
JP = 128 // NUM_THETAS
SBJ = SB * JP


def _repeat_mat():
    row = lax.broadcasted_iota(jnp.int32, (NUM_THETAS, SB), 0)
    col = lax.broadcasted_iota(jnp.int32, (NUM_THETAS, SB), 1)
    return (col % NUM_THETAS == row).astype(jnp.float32)


def _node_body(x_ref, v_ref, b_ref, lin_ref, nh_ref, acc_ref, st_ref):
    nh = jnp.dot(x_ref[...], v_ref[...], preferred_element_type=jnp.float32)
    nh_ref[0:N_NODES, :] = nh
    nh_ref[N_NODES:N_PAD, :] = jnp.zeros((N_PAD - N_NODES, NUM_THETAS),
                                         jnp.float32)
    vals = jnp.dot(nh, _repeat_mat(),
                   preferred_element_type=jnp.float32)
    sig = 1.0 / (1.0 + jnp.exp(SCALE * (vals - lin_ref[...])))
    iota_g = lax.broadcasted_iota(jnp.int32, (1, NUM_GRAPHS), 1)
    oh = (b_ref[...] == iota_g).astype(jnp.float32)
    acc_ref[...] = lax.dot_general(
        oh, sig, (((0,), (0,)), ((), ())),
        preferred_element_type=jnp.float32)
    le = (b_ref[...] <= iota_g).astype(jnp.float32)
    ends = jnp.sum(le, axis=0, keepdims=True)
    counts = jnp.sum(oh, axis=0, keepdims=True)
    st_ref[...] = jnp.concatenate([ends - counts, ends], axis=0)


@functools.cache
def _build_sc_gather():
    mesh = plsc.VectorSubcoreMesh(core_axis_name="c", subcore_axis_name="s")

    @functools.partial(
        pl.kernel,
        mesh=mesh,
        compiler_params=pltpu.CompilerParams(use_tc_tiling_on_sc=False),
        out_type=jax.ShapeDtypeStruct((E_PAD // JP, 128), jnp.float32),
        scratch_types=[
            pltpu.VMEM((2, CHUNK), jnp.int32),
            pltpu.VMEM((2, CHUNK), jnp.int32),
            pltpu.VMEM((2, CHUNK, 16), jnp.float32),
            pltpu.VMEM((2, CHUNK, 16), jnp.float32),
            pltpu.VMEM((CHUNK // JP, 128), jnp.float32),
            pltpu.SemaphoreType.DMA,
            pltpu.SemaphoreType.DMA,
            pltpu.SemaphoreType.DMA,
            pltpu.SemaphoreType.DMA,
        ],
    )
    def _sc_gather(u_hbm, w_hbm, nh_hbm, eh_hbm,
                   iu_v, iw_v, ru_v, rw_v, fl_v, su0, sw0, su1, sw1):
        cid = lax.axis_index("c")
        sid = lax.axis_index("s")
        sems = ((su0, sw0), (su1, sw1))

        def run(base, nround):
            def fire(r, slot):
                off = base + r * CHUNK
                pltpu.sync_copy(u_hbm.at[pl.ds(off, CHUNK)], iu_v.at[slot])
                pltpu.sync_copy(w_hbm.at[pl.ds(off, CHUNK)], iw_v.at[slot])
                cu = pltpu.async_copy(nh_hbm.at[iu_v.at[slot]],
                                      ru_v.at[slot], sems[slot][0])
                cw = pltpu.async_copy(nh_hbm.at[iw_v.at[slot]],
                                      rw_v.at[slot], sems[slot][1])
                return cu, cw

            pend = fire(0, 0)
            for r in range(nround):
                slot = r % 2
                cu, cw = pend
                if r + 1 < nround:
                    nxt = fire(r + 1, (r + 1) % 2)
                cu.wait()
                cw.wait()

                def _mx(i, c, _slot=slot):
                    for k in range(JP):
                        j = i * JP + k
                        fl_v[i, pl.ds(k * NUM_THETAS, NUM_THETAS)] = (
                            jnp.maximum(ru_v[_slot, j], rw_v[_slot, j]))
                    return c

                lax.fori_loop(0, CHUNK // JP, _mx, 0)
                row = (base + r * CHUNK) // JP
                pltpu.sync_copy(fl_v, eh_hbm.at[pl.ds(row, CHUNK // JP)])
                if r + 1 < nround:
                    pend = nxt

        pair = sid * (2 * EW)

        @pl.when(cid == 0)
        def _():
            run(pair, 2 * NROUND - 2)

        @pl.when(cid != 0)
        def _():
            run(pair + (2 * NROUND - 2) * CHUNK, 2)

    return _sc_gather


def _edge_body(nacc_ref, eh_ref, u8_ref, stp_ref, lin16_ref, out_ref,
               acc_ref):
    i32 = jnp.int32
    f32 = jnp.float32
    m2 = eh_ref[...]
    pieces = [
        1.0 / (1.0 + jnp.exp(SCALE * (m2 - lin16_ref[0, s])))
        for s in range(BUMP_STEPS)
    ]
    sigP = jnp.concatenate(pieces, axis=1)
    u8f = u8_ref[...]
    st0 = stp_ref[0:1, 0:NUM_GRAPHS]
    st1 = stp_ref[1:2, 0:NUM_GRAPHS]
    ohs = []
    for j in range(JP):
        uj = u8f[:, j:j + 1]
        ohs.append(((uj >= st0 - 0.5) & (uj < st1 - 0.5)).astype(f32))
    ohp = jnp.concatenate(ohs, axis=1)
    c = lax.dot_general(ohp, sigP, (((0,), (0,)), ((), ())),
                        preferred_element_type=f32)

    @pl.when(pl.program_id(0) == 0)
    def _():
        acc_ref[...] = c

    @pl.when(pl.program_id(0) != 0)
    def _():
        acc_ref[...] = acc_ref[...] + c

    @pl.when(pl.program_id(0) == pl.num_programs(0) - 1)
    def _():
        cols = []
        for s in range(BUMP_STEPS):
            blk = acc_ref[0:NUM_GRAPHS,
                          s * 128:s * 128 + NUM_THETAS]
            for j in range(1, JP):
                blk = blk + acc_ref[j * NUM_GRAPHS:(j + 1) * NUM_GRAPHS,
                                    s * 128 + j * NUM_THETAS:
                                    s * 128 + (j + 1) * NUM_THETAS]
            cols.append(blk)
        out_ref[...] = nacc_ref[...] - jnp.concatenate(cols, axis=1)


def kernel(x, edge_index, batch, v, lin):
    f32 = jnp.float32
    i32 = jnp.int32
    ei_pad = jnp.concatenate(
        [edge_index.astype(i32),
         jnp.full((2, E_PAD - N_EDGES), N_NODES, i32)], axis=1)
    u_flat = ei_pad[0]
    w_flat = ei_pad[1]
    linrep = jnp.repeat(lin.reshape(BUMP_STEPS).astype(f32),
                        NUM_THETAS).reshape(1, SB)
    lin_p = jnp.concatenate([linrep] * JP, axis=1)

    nh_ext, nodeacc, st = pl.pallas_call(
        _node_body,
        out_shape=[
            jax.ShapeDtypeStruct((N_PAD, NUM_THETAS), f32),
            jax.ShapeDtypeStruct((NUM_GRAPHS, SB), f32),
            jax.ShapeDtypeStruct((2, NUM_GRAPHS), f32),
        ],
    )(x, v, batch.reshape(N_NODES, 1), linrep)

    eh = _build_sc_gather()(u_flat, w_flat, nh_ext)

    u8 = u_flat.astype(f32).reshape(E_PAD // JP, JP)
    stp = jnp.tile(st, (1, JP))
    lin16 = lin.reshape(1, BUMP_STEPS).astype(f32)
    out = pl.pallas_call(
        _edge_body,
        grid=(E_PAD // BE,),
        in_specs=[
            pl.BlockSpec((NUM_GRAPHS, SB), lambda i: (0, 0)),
            pl.BlockSpec((BEP, 128), lambda i: (i, 0)),
            pl.BlockSpec((BE // JP, JP), lambda i: (i, 0)),
            pl.BlockSpec((2, SB), lambda i: (0, 0)),
            pl.BlockSpec((1, BUMP_STEPS), lambda i: (0, 0)),
        ],
        out_specs=pl.BlockSpec((NUM_GRAPHS, SB), lambda i: (0, 0)),
        out_shape=jax.ShapeDtypeStruct((NUM_GRAPHS, SB), f32),
        scratch_shapes=[pltpu.VMEM((JP * NUM_GRAPHS, SBJ), f32)],
    )(nodeacc, eh, u8, stp, lin16)

    return out.reshape(NUM_GRAPHS, BUMP_STEPS, NUM_THETAS)
